# baseline (speedup 1.0000x reference)
import functools

import jax
import jax.numpy as jnp
from jax import lax
from jax.experimental import pallas as pl
from jax.experimental.pallas import tpu as pltpu
from jax.experimental.pallas import tpu_sc as plsc

_N = 100000
_H = 128

_NC, _NS = 2, 16
_NW = _NC * _NS
_BPW = 6400
_CH = 128
_NCHUNK = _BPW // _CH
_NBUF = 5
_NGRP = _NCHUNK // _NBUF
_BPAD = _NW * _BPW


@functools.cache
def _make_gather_rows():
    sc_mesh = plsc.VectorSubcoreMesh(
        core_axis_name="c", subcore_axis_name="s", num_cores=_NC, num_subcores=_NS
    )

    @functools.partial(
        pl.kernel,
        out_type=jax.ShapeDtypeStruct((_BPAD, _H), jnp.float32),
        mesh=sc_mesh,
        scratch_types=[
            pltpu.VMEM((_BPW,), jnp.int32),
            *([pltpu.VMEM((_CH, _H), jnp.float32)] * _NBUF),
            *([pltpu.SemaphoreType.DMA] * (2 * _NBUF)),
        ],
    )
    def gather_rows(h_hbm, idx_hbm, out_hbm, idx_v, *scr):
        bufs = scr[:_NBUF]
        gsem = scr[_NBUF:2 * _NBUF]
        ssem = scr[2 * _NBUF:]
        wid = lax.axis_index("s") * _NC + lax.axis_index("c")
        base = wid * _BPW
        pltpu.sync_copy(idx_hbm.at[pl.ds(base, _BPW)], idx_v)

        def gather(c, b):
            return pltpu.make_async_copy(
                h_hbm.at[idx_v.at[pl.ds(c * _CH, _CH)]], bufs[b], gsem[b])

        def scatter(c, b):
            return pltpu.make_async_copy(
                bufs[b], out_hbm.at[pl.ds(base + c * _CH, _CH)], ssem[b])

        for b in range(_NBUF):
            gather(b, b).start()

        def group(g, carry):
            for b in range(_NBUF):
                c = g * _NBUF + b
                gather(c, b).wait()
                scatter(c, b).start()
            for b in range(_NBUF):
                c = g * _NBUF + b

                @pl.when(g + 1 < _NGRP)
                def _():
                    scatter(c, b).wait()
                    gather(c + _NBUF, b).start()

            return carry

        lax.fori_loop(0, _NGRP, group, 0)
        for b in range(_NBUF):
            scatter((_NGRP - 1) * _NBUF + b, b).wait()

    return gather_rows


_BLK = 4000
_GRID = _N // _BLK

_DN = (((1,), (1,)), ((), ()))


def _dense_body(x_ref, hcat_ref, w_ref, u_ref, b_ref, u2_ref, out_ref):
    f32 = jnp.float32
    hcat = hcat_ref[...]
    uo = lax.dot_general(x_ref[...], w_ref[...], _DN, preferred_element_type=f32)
    uo += lax.dot_general(hcat, u_ref[...], _DN, preferred_element_type=f32)
    uo += b_ref[...]
    u = jax.nn.sigmoid(uo[:, :_H])
    o = jnp.tanh(uo[:, _H:])
    h_agg = lax.dot_general(hcat, u2_ref[...], _DN, preferred_element_type=f32)
    out_ref[...] = o * u + (1.0 - u) * h_agg


def _dense(x, hcat, w_uo, u_uo, b_uo, u_2):
    full = lambda shape: pl.BlockSpec(shape, lambda i: (0, 0))
    return pl.pallas_call(
        _dense_body,
        grid=(_GRID,),
        in_specs=[
            pl.BlockSpec((_BLK, _H), lambda i: (i, 0)),
            pl.BlockSpec((_BLK, 2 * _H), lambda i: (i, 0)),
            full((2 * _H, _H)),
            full((2 * _H, 2 * _H)),
            full((1, 2 * _H)),
            full((_H, 2 * _H)),
        ],
        out_specs=pl.BlockSpec((_BLK, _H), lambda i: (i, 0)),
        out_shape=jax.ShapeDtypeStruct((_N, _H), jnp.float32),
    )(x, hcat, w_uo, u_uo, b_uo, u_2)


def kernel(x, h, child_idx, W_ruo, U_ruo, b_ruo, U_u2):
    pad = jnp.arange(_BPAD - 2 * _N, dtype=jnp.int32) * 17 % _N
    idx = jnp.concatenate([child_idx.reshape(-1), pad])
    gathered = _make_gather_rows()(h, idx)
    hcat = gathered.reshape(_BPAD // 2, 2 * _H)[:_N]
    return _dense(x, hcat, W_ruo[_H:], U_ruo[_H:], b_ruo[:, _H:], U_u2)


# speedup vs baseline: 2.1610x; 2.1610x over previous
import functools

import jax
import jax.numpy as jnp
from jax import lax
from jax.experimental import pallas as pl
from jax.experimental.pallas import tpu as pltpu
from jax.experimental.pallas import tpu_sc as plsc

_N = 100000
_H = 128

_NC, _NS = 2, 16
_NW = _NC * _NS
_BPW = 6400
_CH = 128
_NCHUNK = _BPW // _CH
_NBUF = 5
_NGRP = _NCHUNK // _NBUF
_BPAD = _NW * _BPW


@functools.cache
def _make_gather_rows():
    sc_mesh = plsc.VectorSubcoreMesh(
        core_axis_name="c", subcore_axis_name="s", num_cores=_NC, num_subcores=_NS
    )

    @functools.partial(
        pl.kernel,
        out_type=jax.ShapeDtypeStruct((_BPAD, _H), jnp.float32),
        mesh=sc_mesh,
        scratch_types=[
            pltpu.VMEM((_BPW,), jnp.int32),
            *([pltpu.VMEM((_CH, _H), jnp.float32)] * _NBUF),
            *([pltpu.SemaphoreType.DMA] * (2 * _NBUF)),
        ],
    )
    def gather_rows(h_hbm, idx_hbm, out_hbm, idx_v, *scr):
        bufs = scr[:_NBUF]
        gsem = scr[_NBUF:2 * _NBUF]
        ssem = scr[2 * _NBUF:]
        wid = lax.axis_index("s") * _NC + lax.axis_index("c")
        base = wid * _BPW
        pltpu.sync_copy(idx_hbm.at[pl.ds(base, _BPW)], idx_v)

        def gather(c, b):
            return pltpu.make_async_copy(
                h_hbm.at[idx_v.at[pl.ds(c * _CH, _CH)]], bufs[b], gsem[b])

        def scatter(c, b):
            return pltpu.make_async_copy(
                bufs[b], out_hbm.at[pl.ds(base + c * _CH, _CH)], ssem[b])

        for b in range(_NBUF):
            gather(b, b).start()

        def group(g, carry):
            for b in range(_NBUF):
                c = g * _NBUF + b
                gather(c, b).wait()
                scatter(c, b).start()
            for b in range(_NBUF):
                c = g * _NBUF + b

                @pl.when(g + 1 < _NGRP)
                def _():
                    scatter(c, b).wait()
                    gather(c + _NBUF, b).start()

            return carry

        lax.fori_loop(0, _NGRP, group, 0)
        for b in range(_NBUF):
            scatter((_NGRP - 1) * _NBUF + b, b).wait()

    return gather_rows


_BLK = 4000
_GRID = _N // _BLK
_C1_OFF = _N // _BLK

_DN = (((1,), (1,)), ((), ()))


def _dense_body(x_ref, hc0_ref, hc1_ref, w_ref, u0_ref, u1_ref, b_ref,
                u20_ref, u21_ref, out_ref):
    f32 = jnp.float32
    hc0 = hc0_ref[...]
    hc1 = hc1_ref[...]
    uo = lax.dot_general(x_ref[...], w_ref[...], _DN, preferred_element_type=f32)
    uo += lax.dot_general(hc0, u0_ref[...], _DN, preferred_element_type=f32)
    uo += lax.dot_general(hc1, u1_ref[...], _DN, preferred_element_type=f32)
    uo += b_ref[...]
    u = jax.nn.sigmoid(uo[:, :_H])
    o = jnp.tanh(uo[:, _H:])
    h_agg = lax.dot_general(hc0, u20_ref[...], _DN, preferred_element_type=f32)
    h_agg += lax.dot_general(hc1, u21_ref[...], _DN, preferred_element_type=f32)
    out_ref[...] = o * u + (1.0 - u) * h_agg


def _dense(x, gathered, w_uo, u0, u1, b_uo, u20, u21):
    full = lambda shape: pl.BlockSpec(shape, lambda i: (0, 0))
    return pl.pallas_call(
        _dense_body,
        grid=(_GRID,),
        in_specs=[
            pl.BlockSpec((_BLK, _H), lambda i: (i, 0)),
            pl.BlockSpec((_BLK, _H), lambda i: (i, 0)),
            pl.BlockSpec((_BLK, _H), lambda i: (i + _C1_OFF, 0)),
            full((2 * _H, _H)),
            full((2 * _H, _H)),
            full((2 * _H, _H)),
            full((1, 2 * _H)),
            full((_H, _H)),
            full((_H, _H)),
        ],
        out_specs=pl.BlockSpec((_BLK, _H), lambda i: (i, 0)),
        out_shape=jax.ShapeDtypeStruct((_N, _H), jnp.float32),
    )(x, gathered, gathered, w_uo, u0, u1, b_uo, u20, u21)


def kernel(x, h, child_idx, W_ruo, U_ruo, b_ruo, U_u2):
    pad = jnp.arange(_BPAD - 2 * _N, dtype=jnp.int32) * 17 % _N
    idx = jnp.concatenate([child_idx[:, 0], child_idx[:, 1], pad])
    gathered = _make_gather_rows()(h, idx)
    return _dense(x, gathered,
                  W_ruo[_H:],
                  U_ruo[_H:, :_H],
                  U_ruo[_H:, _H:],
                  b_ruo[:, _H:],
                  U_u2[:, :_H],
                  U_u2[:, _H:])
